# baseline (speedup 1.0000x reference)
import functools

import jax
import jax.numpy as jnp
import numpy as np
from jax.experimental import pallas as pl
from jax.experimental.pallas import tpu as pltpu
from jax.sharding import PartitionSpec as P

_K = 8
_NB = 4
_NEG = -1e30
_VBLK = 256


def _knn1_body(vf_ref, pf_ref, ptsT_ref, vb_ref, pb_ref, vtx_ref, out_ref):
    vf = vf_ref[...]
    pf = pf_ref[...]
    rows, p = vf.shape[0], pf.shape[0]

    nsq = jnp.sum(pf * pf, axis=1, keepdims=True)
    inv = 1.0 / (jnp.sqrt(nsq) + 1e-12)
    pfn = pf * inv

    qsq = jnp.sum(vf * vf, axis=1, keepdims=True)
    qn = vf * (1.0 / (jnp.sqrt(qsq) + 1e-12))

    dn = (((1,), (1,)), ((), ()))
    S = jax.lax.dot_general(vf, pf, dn, precision=jax.lax.Precision.HIGHEST,
                            preferred_element_type=jnp.float32)
    C = jax.lax.dot_general(qn, pfn, dn,
                            preferred_element_type=jnp.float32)

    mask = vb_ref[...] == pb_ref[...]
    C = jnp.where(mask, C, _NEG)

    iota = jax.lax.broadcasted_iota(jnp.int32, (rows, p), 1)
    vtx = vtx_ref[...]

    for _ in range(_K):
        m = jnp.max(C, axis=1, keepdims=True)
        idx = jnp.min(jnp.where(C == m, iota, p), axis=1, keepdims=True)
        C = jnp.where(iota == idx, -jnp.inf, C)

    W = jnp.where(C == -jnp.inf, S, 0.0)
    E = jax.lax.dot_general(W, ptsT_ref[...], (((1,), (1,)), ((), ())),
                            precision=jax.lax.Precision.HIGHEST,
                            preferred_element_type=jnp.float32)
    den = E[:, 3:4]
    out_ref[:, 0:1] = E[:, 0:1] / den - vtx[:, 0:1]
    out_ref[:, 1:2] = E[:, 1:2] / den - vtx[:, 1:2]
    out_ref[:, 2:3] = E[:, 2:3] / den - vtx[:, 2:3]


def _pv_from(logits, batch, mx, mn):
    s = jax.nn.sigmoid(logits)
    out = jnp.zeros_like(s)
    for b in range(_NB):
        out = jnp.where(batch == b, (s - mn[b]) / (mx[b] - mn[b]), out)
    return out


def _knn2_body(vf_ref, vfull_ref, vb_ref, vbr_ref, vl_ref, vlr_ref,
               flow_ref, flowT_ref, out_ref):
    vf = vf_ref[...]
    vfull = vfull_ref[...]
    rows, v = vf.shape[0], vfull.shape[0]

    vbr = vbr_ref[...]
    sr = jax.nn.sigmoid(vlr_ref[...])
    mx, mn = [], []
    for b in range(_NB):
        inb = vbr == b
        mx.append(jnp.max(jnp.where(inb, sr, -jnp.inf), axis=1, keepdims=True))
        mn.append(jnp.min(jnp.where(inb, sr, jnp.inf), axis=1, keepdims=True))
    pv_row = _pv_from(vlr_ref[...], vbr, mx, mn)
    vis_row = pv_row >= 0.5
    pv_blk = _pv_from(vl_ref[...], vb_ref[...], mx, mn)
    vis_blk = pv_blk >= 0.5

    nsq = jnp.sum(vfull * vfull, axis=1, keepdims=True)
    inv = 1.0 / (jnp.sqrt(nsq) + 1e-12)
    vfn = vfull * inv

    qsq = jnp.sum(vf * vf, axis=1, keepdims=True)
    qn = vf * (1.0 / (jnp.sqrt(qsq) + 1e-12))

    dn = (((1,), (1,)), ((), ()))
    S = jax.lax.dot_general(vf, vfull, dn, precision=jax.lax.Precision.HIGHEST,
                            preferred_element_type=jnp.float32)
    C = jax.lax.dot_general(qn, vfn, dn,
                            preferred_element_type=jnp.float32)

    mask = (vb_ref[...] == vbr) & vis_row
    C = jnp.where(mask, C, _NEG)

    iota = jax.lax.broadcasted_iota(jnp.int32, (rows, v), 1)

    for _ in range(_K):
        m = jnp.max(C, axis=1, keepdims=True)
        idx = jnp.min(jnp.where(C == m, iota, v), axis=1, keepdims=True)
        C = jnp.where(iota == idx, -jnp.inf, C)

    W = jnp.where(C == -jnp.inf, S, 0.0)
    E = jax.lax.dot_general(W, flowT_ref[...], (((1,), (1,)), ((), ())),
                            precision=jax.lax.Precision.HIGHEST,
                            preferred_element_type=jnp.float32)
    den = E[:, 3:4]

    flow = flow_ref[...]
    out_ref[:, 0:1] = jnp.where(vis_blk, flow[:, 0:1], E[:, 0:1] / den)
    out_ref[:, 1:2] = jnp.where(vis_blk, flow[:, 1:2], E[:, 1:2] / den)
    out_ref[:, 2:3] = jnp.where(vis_blk, flow[:, 2:3], E[:, 2:3] / den)
    out_ref[:, 3:4] = pv_blk


_PARAMS = pltpu.CompilerParams(dimension_semantics=("arbitrary",))
_BLK = lambda i: (i, 0)
_FULL = lambda i: (0, 0)


def _phase1(vf_loc, pf, ptsT, vb_col, pb_row, vtx_loc):
    vloc, d = vf_loc.shape
    p = pf.shape[0]
    nblk = vloc // _VBLK
    return pl.pallas_call(
        _knn1_body,
        grid=(nblk,),
        in_specs=[
            pl.BlockSpec((_VBLK, d), _BLK),
            pl.BlockSpec((p, d), _FULL),
            pl.BlockSpec((4, p), _FULL),
            pl.BlockSpec((_VBLK, 1), _BLK),
            pl.BlockSpec((1, p), _FULL),
            pl.BlockSpec((_VBLK, 3), _BLK),
        ],
        out_specs=pl.BlockSpec((_VBLK, 3), _BLK),
        out_shape=jax.ShapeDtypeStruct((vloc, 3), jnp.float32),
        compiler_params=_PARAMS,
    )(vf_loc, pf, ptsT, vb_col, pb_row, vtx_loc)


def _phase2(vf_loc, vf_full, vb_col, vb_row, vl_col, vl_row, flow_loc, flowT):
    vloc, d = vf_loc.shape
    v = vf_full.shape[0]
    nblk = vloc // _VBLK
    return pl.pallas_call(
        _knn2_body,
        grid=(nblk,),
        in_specs=[
            pl.BlockSpec((_VBLK, d), _BLK),
            pl.BlockSpec((v, d), _FULL),
            pl.BlockSpec((_VBLK, 1), _BLK),
            pl.BlockSpec((1, v), _FULL),
            pl.BlockSpec((_VBLK, 1), _BLK),
            pl.BlockSpec((1, v), _FULL),
            pl.BlockSpec((_VBLK, 3), _BLK),
            pl.BlockSpec((4, v), _FULL),
        ],
        out_specs=pl.BlockSpec((_VBLK, 4), _BLK),
        out_shape=jax.ShapeDtypeStruct((vloc, 4), jnp.float32),
        compiler_params=_PARAMS,
    )(vf_loc, vf_full, vb_col, vb_row, vl_col, vl_row, flow_loc, flowT)


def _ones_row(a):
    return jnp.concatenate([a.T, jnp.ones((1, a.shape[0]), jnp.float32)])


def _run_single(vtx, pts, vf, pf, vl, vb, pb):
    flow = _phase1(vf, pf, _ones_row(pts), vb[:, None], pb[None, :], vtx)
    return _phase2(vf, vf, vb[:, None], vb[None, :], vl, vl.T, flow,
                   _ones_row(flow))


def _build(n_dev):
    if n_dev < 2:
        return jax.jit(_run_single)

    mesh = jax.sharding.Mesh(np.array(jax.devices()[:2]), ("x",))

    def _body(vtx, pts, vf, pf, vl, vb, pb):
        i = jax.lax.axis_index("x")
        half = vf.shape[0] // 2
        sl = lambda a: jax.lax.dynamic_slice_in_dim(a, i * half, half, axis=0)
        vf_h, vtx_h, vl_h, vb_h = sl(vf), sl(vtx), sl(vl), sl(vb)
        flow_h = _phase1(vf_h, pf, _ones_row(pts), vb_h[:, None],
                         pb[None, :], vtx_h)
        flow_f = jax.lax.all_gather(flow_h, "x", axis=0, tiled=True)
        return _phase2(vf_h, vf, vb_h[:, None], vb[None, :], vl_h, vl.T,
                       flow_h, _ones_row(flow_f))

    try:
        from jax.experimental.shard_map import shard_map
    except ImportError:
        shard_map = jax.shard_map

    sharded = shard_map(
        _body, mesh=mesh,
        in_specs=(P(), P(), P(), P(), P(), P(), P()),
        out_specs=P("x"),
        check_rep=False,
    )
    return jax.jit(sharded)


_CACHE = {}


def kernel(vtx, pts, vtx_feature, pts_feature, vis_logits, vtx_batch, pts_batch):
    n_dev = len(jax.devices())
    fn = _CACHE.get(n_dev)
    if fn is None:
        fn = _CACHE[n_dev] = _build(n_dev)
    return fn(vtx, pts, vtx_feature, pts_feature, vis_logits,
              vtx_batch, pts_batch)


# speedup vs baseline: 1.3719x; 1.3719x over previous
import functools

import jax
import jax.numpy as jnp
import numpy as np
from jax.experimental import pallas as pl
from jax.experimental.pallas import tpu as pltpu
from jax.sharding import PartitionSpec as P

_K = 8
_NB = 4
_NEG = -1e30
_VBLK = 512


def _knn1_body(vf_ref, pf_ref, ptsT_ref, vb_ref, pb_ref, vtx_ref, out_ref):
    vf = vf_ref[...]
    pf = pf_ref[...]
    rows, p = vf.shape[0], pf.shape[0]

    nsq = jnp.sum(pf * pf, axis=1, keepdims=True)
    inv = 1.0 / (jnp.sqrt(nsq) + 1e-12)
    pfn = pf * inv

    qsq = jnp.sum(vf * vf, axis=1, keepdims=True)
    qn = vf * (1.0 / (jnp.sqrt(qsq) + 1e-12))

    dn = (((1,), (1,)), ((), ()))
    S = jax.lax.dot_general(vf, pf, dn, precision=jax.lax.Precision.HIGHEST,
                            preferred_element_type=jnp.float32)
    C = jax.lax.dot_general(qn, pfn, dn,
                            preferred_element_type=jnp.float32)

    mask = vb_ref[...] == pb_ref[...]
    C = jnp.where(mask, C, _NEG)

    iota = jax.lax.broadcasted_iota(jnp.int32, (rows, p), 1)
    vtx = vtx_ref[...]

    for _ in range(_K):
        m = jnp.max(C, axis=1, keepdims=True)
        idx = jnp.min(jnp.where(C == m, iota, p), axis=1, keepdims=True)
        C = jnp.where(iota == idx, -jnp.inf, C)

    W = jnp.where(C == -jnp.inf, S, 0.0)
    den = jnp.sum(W, axis=1, keepdims=True)
    num0 = jnp.sum(W * ptsT_ref[0:1, :], axis=1, keepdims=True)
    num1 = jnp.sum(W * ptsT_ref[1:2, :], axis=1, keepdims=True)
    num2 = jnp.sum(W * ptsT_ref[2:3, :], axis=1, keepdims=True)

    out_ref[:, 0:1] = num0 / den - vtx[:, 0:1]
    out_ref[:, 1:2] = num1 / den - vtx[:, 1:2]
    out_ref[:, 2:3] = num2 / den - vtx[:, 2:3]


def _pv_from(logits, batch, mx, mn):
    s = jax.nn.sigmoid(logits)
    out = jnp.zeros_like(s)
    for b in range(_NB):
        out = jnp.where(batch == b, (s - mn[b]) / (mx[b] - mn[b]), out)
    return out


def _knn2_body(vf_ref, vfull_ref, vb_ref, vbr_ref, vl_ref, vlr_ref,
               flow_ref, flowT_ref, out_ref):
    vf = vf_ref[...]
    vfull = vfull_ref[...]
    rows, v = vf.shape[0], vfull.shape[0]

    vbr = vbr_ref[...]
    sr = jax.nn.sigmoid(vlr_ref[...])
    mx, mn = [], []
    for b in range(_NB):
        inb = vbr == b
        mx.append(jnp.max(jnp.where(inb, sr, -jnp.inf), axis=1, keepdims=True))
        mn.append(jnp.min(jnp.where(inb, sr, jnp.inf), axis=1, keepdims=True))
    pv_row = _pv_from(vlr_ref[...], vbr, mx, mn)
    vis_row = pv_row >= 0.5
    pv_blk = _pv_from(vl_ref[...], vb_ref[...], mx, mn)
    vis_blk = pv_blk >= 0.5

    nsq = jnp.sum(vfull * vfull, axis=1, keepdims=True)
    inv = 1.0 / (jnp.sqrt(nsq) + 1e-12)
    vfn = vfull * inv

    qsq = jnp.sum(vf * vf, axis=1, keepdims=True)
    qn = vf * (1.0 / (jnp.sqrt(qsq) + 1e-12))

    dn = (((1,), (1,)), ((), ()))
    S = jax.lax.dot_general(vf, vfull, dn, precision=jax.lax.Precision.HIGHEST,
                            preferred_element_type=jnp.float32)
    C = jax.lax.dot_general(qn, vfn, dn,
                            preferred_element_type=jnp.float32)

    mask = (vb_ref[...] == vbr) & vis_row
    C = jnp.where(mask, C, _NEG)

    iota = jax.lax.broadcasted_iota(jnp.int32, (rows, v), 1)

    for _ in range(_K):
        m = jnp.max(C, axis=1, keepdims=True)
        idx = jnp.min(jnp.where(C == m, iota, v), axis=1, keepdims=True)
        C = jnp.where(iota == idx, -jnp.inf, C)

    W = jnp.where(C == -jnp.inf, S, 0.0)
    den = jnp.sum(W, axis=1, keepdims=True)
    num0 = jnp.sum(W * flowT_ref[0:1, :], axis=1, keepdims=True)
    num1 = jnp.sum(W * flowT_ref[1:2, :], axis=1, keepdims=True)
    num2 = jnp.sum(W * flowT_ref[2:3, :], axis=1, keepdims=True)

    flow = flow_ref[...]
    out_ref[:, 0:1] = jnp.where(vis_blk, flow[:, 0:1], num0 / den)
    out_ref[:, 1:2] = jnp.where(vis_blk, flow[:, 1:2], num1 / den)
    out_ref[:, 2:3] = jnp.where(vis_blk, flow[:, 2:3], num2 / den)
    out_ref[:, 3:4] = pv_blk


_PARAMS = pltpu.CompilerParams(dimension_semantics=("arbitrary",))
_BLK = lambda i: (i, 0)
_FULL = lambda i: (0, 0)


def _phase1(vf_loc, pf, ptsT, vb_col, pb_row, vtx_loc):
    vloc, d = vf_loc.shape
    p = pf.shape[0]
    nblk = vloc // _VBLK
    return pl.pallas_call(
        _knn1_body,
        grid=(nblk,),
        in_specs=[
            pl.BlockSpec((_VBLK, d), _BLK),
            pl.BlockSpec((p, d), _FULL),
            pl.BlockSpec((4, p), _FULL),
            pl.BlockSpec((_VBLK, 1), _BLK),
            pl.BlockSpec((1, p), _FULL),
            pl.BlockSpec((_VBLK, 3), _BLK),
        ],
        out_specs=pl.BlockSpec((_VBLK, 3), _BLK),
        out_shape=jax.ShapeDtypeStruct((vloc, 3), jnp.float32),
        compiler_params=_PARAMS,
    )(vf_loc, pf, ptsT, vb_col, pb_row, vtx_loc)


def _phase2(vf_loc, vf_full, vb_col, vb_row, vl_col, vl_row, flow_loc, flowT):
    vloc, d = vf_loc.shape
    v = vf_full.shape[0]
    nblk = vloc // _VBLK
    return pl.pallas_call(
        _knn2_body,
        grid=(nblk,),
        in_specs=[
            pl.BlockSpec((_VBLK, d), _BLK),
            pl.BlockSpec((v, d), _FULL),
            pl.BlockSpec((_VBLK, 1), _BLK),
            pl.BlockSpec((1, v), _FULL),
            pl.BlockSpec((_VBLK, 1), _BLK),
            pl.BlockSpec((1, v), _FULL),
            pl.BlockSpec((_VBLK, 3), _BLK),
            pl.BlockSpec((4, v), _FULL),
        ],
        out_specs=pl.BlockSpec((_VBLK, 4), _BLK),
        out_shape=jax.ShapeDtypeStruct((vloc, 4), jnp.float32),
        compiler_params=_PARAMS,
    )(vf_loc, vf_full, vb_col, vb_row, vl_col, vl_row, flow_loc, flowT)


def _ones_row(a):
    return jnp.concatenate([a.T, jnp.ones((1, a.shape[0]), jnp.float32)])


def _run_single(vtx, pts, vf, pf, vl, vb, pb):
    flow = _phase1(vf, pf, _ones_row(pts), vb[:, None], pb[None, :], vtx)
    return _phase2(vf, vf, vb[:, None], vb[None, :], vl, vl.T, flow,
                   _ones_row(flow))


def _build(n_dev):
    if n_dev < 2:
        return jax.jit(_run_single)

    mesh = jax.sharding.Mesh(np.array(jax.devices()[:2]), ("x",))

    def _body(vtx, pts, vf, pf, vl, vb, pb):
        i = jax.lax.axis_index("x")
        half = vf.shape[0] // 2
        sl = lambda a: jax.lax.dynamic_slice_in_dim(a, i * half, half, axis=0)
        vf_h, vtx_h, vl_h, vb_h = sl(vf), sl(vtx), sl(vl), sl(vb)
        flow_h = _phase1(vf_h, pf, _ones_row(pts), vb_h[:, None],
                         pb[None, :], vtx_h)
        flow_f = jax.lax.all_gather(flow_h, "x", axis=0, tiled=True)
        return _phase2(vf_h, vf, vb_h[:, None], vb[None, :], vl_h, vl.T,
                       flow_h, _ones_row(flow_f))

    try:
        from jax.experimental.shard_map import shard_map
    except ImportError:
        shard_map = jax.shard_map

    sharded = shard_map(
        _body, mesh=mesh,
        in_specs=(P(), P(), P(), P(), P(), P(), P()),
        out_specs=P("x"),
        check_rep=False,
    )
    return jax.jit(sharded)


_CACHE = {}


def kernel(vtx, pts, vtx_feature, pts_feature, vis_logits, vtx_batch, pts_batch):
    n_dev = len(jax.devices())
    fn = _CACHE.get(n_dev)
    if fn is None:
        fn = _CACHE[n_dev] = _build(n_dev)
    return fn(vtx, pts, vtx_feature, pts_feature, vis_logits,
              vtx_batch, pts_batch)


# speedup vs baseline: 1.7226x; 1.2556x over previous
import jax
import jax.numpy as jnp
import numpy as np
from jax.experimental import pallas as pl
from jax.experimental.pallas import tpu as pltpu
from jax.sharding import PartitionSpec as P

_K = 8
_NB = 4
_NEG = -1e30
_VBLK = 256
_T = 512
_BIG = 1 << 30


def _scan_topk(cbuf, lo, hi, rows):
    iota_t = jax.lax.broadcasted_iota(jnp.int32, (rows, _T), 1)
    for _ in range(_K):
        def maxstep(t, acc):
            return jnp.maximum(acc, jnp.max(cbuf[t], axis=1, keepdims=True))
        m = jax.lax.fori_loop(lo, hi, maxstep,
                              jnp.full((rows, 1), -jnp.inf, jnp.float32))

        def locstep(t, acc):
            io = iota_t + t * _T
            cand = jnp.min(jnp.where(cbuf[t] == m, io, _BIG),
                           axis=1, keepdims=True)
            return jnp.minimum(acc, cand)
        idx = jax.lax.fori_loop(lo, hi, locstep,
                                jnp.full((rows, 1), _BIG, jnp.int32))

        def constep(t, _):
            io = iota_t + t * _T
            cbuf[t] = jnp.where(io == idx, -jnp.inf, cbuf[t])
            return 0
        jax.lax.fori_loop(lo, hi, constep, 0)


def _accumulate(cbuf, sbuf, r4_ref, lo, hi, rows):
    zero = jnp.zeros((rows, 1), jnp.float32)

    def accstep(t, carry):
        d, n0, n1, n2 = carry
        w = jnp.where(cbuf[t] == -jnp.inf, sbuf[t], 0.0)
        r4 = r4_ref[t]
        d = d + jnp.sum(w, axis=1, keepdims=True)
        n0 = n0 + jnp.sum(w * r4[0:1, :], axis=1, keepdims=True)
        n1 = n1 + jnp.sum(w * r4[1:2, :], axis=1, keepdims=True)
        n2 = n2 + jnp.sum(w * r4[2:3, :], axis=1, keepdims=True)
        return (d, n0, n1, n2)

    return jax.lax.fori_loop(lo, hi, accstep, (zero, zero, zero, zero))


_DN = (((1,), (1,)), ((), ()))


def _knn1_body(lo_ref, nt_ref, vf_ref, pf_ref, pts1_ref, vb_ref, pb_ref,
               vtx_ref, out_ref, sbuf, cbuf):
    i = pl.program_id(0)
    lo = lo_ref[i]
    hi = lo + nt_ref[i]

    vf = vf_ref[...]
    rows = vf.shape[0]
    qsq = jnp.sum(vf * vf, axis=1, keepdims=True)
    qn = vf * (1.0 / (jnp.sqrt(qsq) + 1e-12))
    vb = vb_ref[...]

    def init_tile(t, _):
        pft = pf_ref[t]
        nsq = jnp.sum(pft * pft, axis=1, keepdims=True)
        pfn = pft * (1.0 / (jnp.sqrt(nsq) + 1e-12))
        S_t = jax.lax.dot_general(vf, pft, _DN,
                                  precision=jax.lax.Precision.HIGHEST,
                                  preferred_element_type=jnp.float32)
        C_t = jax.lax.dot_general(qn, pfn, _DN,
                                  preferred_element_type=jnp.float32)
        mask = vb == pb_ref[t]
        sbuf[t] = S_t
        cbuf[t] = jnp.where(mask, C_t, _NEG)
        return 0

    jax.lax.fori_loop(lo, hi, init_tile, 0)
    _scan_topk(cbuf, lo, hi, rows)
    den, num0, num1, num2 = _accumulate(cbuf, sbuf, pts1_ref, lo, hi, rows)

    vtx = vtx_ref[...]
    out_ref[:, 0:1] = num0 / den - vtx[:, 0:1]
    out_ref[:, 1:2] = num1 / den - vtx[:, 1:2]
    out_ref[:, 2:3] = num2 / den - vtx[:, 2:3]


def _pv_from(logits, batch, mx, mn):
    s = jax.nn.sigmoid(logits)
    out = jnp.zeros_like(s)
    for b in range(_NB):
        out = jnp.where(batch == b, (s - mn[b]) / (mx[b] - mn[b]), out)
    return out


def _knn2_body(lo_ref, nt_ref, vf_ref, vf3_ref, vb_ref, vbr_ref, vbr3_ref,
               vl_ref, vlr_ref, vlr3_ref, flow_ref, flow13_ref,
               out_ref, sbuf, cbuf):
    i = pl.program_id(0)
    lo = lo_ref[i]
    hi = lo + nt_ref[i]

    vf = vf_ref[...]
    rows = vf.shape[0]

    vbr = vbr_ref[...]
    sr = jax.nn.sigmoid(vlr_ref[...])
    mx, mn = [], []
    for b in range(_NB):
        inb = vbr == b
        mx.append(jnp.max(jnp.where(inb, sr, -jnp.inf), axis=1, keepdims=True))
        mn.append(jnp.min(jnp.where(inb, sr, jnp.inf), axis=1, keepdims=True))
    pv_blk = _pv_from(vl_ref[...], vb_ref[...], mx, mn)
    vis_blk = pv_blk >= 0.5

    qsq = jnp.sum(vf * vf, axis=1, keepdims=True)
    qn = vf * (1.0 / (jnp.sqrt(qsq) + 1e-12))
    vb = vb_ref[...]

    def init_tile(t, _):
        kt = vf3_ref[t]
        nsq = jnp.sum(kt * kt, axis=1, keepdims=True)
        kn = kt * (1.0 / (jnp.sqrt(nsq) + 1e-12))
        S_t = jax.lax.dot_general(vf, kt, _DN,
                                  precision=jax.lax.Precision.HIGHEST,
                                  preferred_element_type=jnp.float32)
        C_t = jax.lax.dot_general(qn, kn, _DN,
                                  preferred_element_type=jnp.float32)
        pv_t = _pv_from(vlr3_ref[t], vbr3_ref[t], mx, mn)
        mask = (vb == vbr3_ref[t]) & (pv_t >= 0.5)
        sbuf[t] = S_t
        cbuf[t] = jnp.where(mask, C_t, _NEG)
        return 0

    jax.lax.fori_loop(lo, hi, init_tile, 0)
    _scan_topk(cbuf, lo, hi, rows)
    den, num0, num1, num2 = _accumulate(cbuf, sbuf, flow13_ref, lo, hi, rows)

    flow = flow_ref[...]
    out_ref[:, 0:1] = jnp.where(vis_blk, flow[:, 0:1], num0 / den)
    out_ref[:, 1:2] = jnp.where(vis_blk, flow[:, 1:2], num1 / den)
    out_ref[:, 2:3] = jnp.where(vis_blk, flow[:, 2:3], num2 / den)
    out_ref[:, 3:4] = pv_blk


_PARAMS = pltpu.CompilerParams(dimension_semantics=("arbitrary",))


def _blk(i, *_):
    return (i, 0)


def _full2(i, *_):
    return (0, 0)


def _full3(i, *_):
    return (0, 0, 0)


def _bounds(q_batch_blocks, key_batch, n_tiles):
    b0 = q_batch_blocks[:, 0]
    b1 = q_batch_blocks[:, -1]
    start = jnp.searchsorted(key_batch, b0, side="left")
    end = jnp.searchsorted(key_batch, b1, side="right")
    lo = (start // _T).astype(jnp.int32)
    hi = ((end + _T - 1) // _T).astype(jnp.int32)
    ok = end > start
    lo = jnp.where(ok, lo, 0)
    nt = jnp.where(ok, jnp.minimum(hi - lo, n_tiles), 1).astype(jnp.int32)
    return lo, nt


def _ones_tiles(a):
    n = a.shape[0]
    r4 = jnp.concatenate([a.T, jnp.ones((1, n), jnp.float32)])
    return r4.reshape(4, n // _T, _T).transpose(1, 0, 2)


def _phase1(vf_loc, pf, pts, vb_loc, pb, vtx_loc):
    vloc, d = vf_loc.shape
    p = pf.shape[0]
    nblk = vloc // _VBLK
    ntile = p // _T
    lo, nt = _bounds(vb_loc.reshape(nblk, _VBLK), pb, ntile)
    grid_spec = pltpu.PrefetchScalarGridSpec(
        num_scalar_prefetch=2,
        grid=(nblk,),
        in_specs=[
            pl.BlockSpec((_VBLK, d), _blk),
            pl.BlockSpec((ntile, _T, d), _full3),
            pl.BlockSpec((ntile, 4, _T), _full3),
            pl.BlockSpec((_VBLK, 1), _blk),
            pl.BlockSpec((ntile, 1, _T), _full3),
            pl.BlockSpec((_VBLK, 3), _blk),
        ],
        out_specs=pl.BlockSpec((_VBLK, 3), _blk),
        scratch_shapes=[
            pltpu.VMEM((ntile, _VBLK, _T), jnp.float32),
            pltpu.VMEM((ntile, _VBLK, _T), jnp.float32),
        ],
    )
    return pl.pallas_call(
        _knn1_body,
        grid_spec=grid_spec,
        out_shape=jax.ShapeDtypeStruct((vloc, 3), jnp.float32),
        compiler_params=_PARAMS,
    )(lo, nt, vf_loc, pf.reshape(ntile, _T, d), _ones_tiles(pts),
      vb_loc[:, None], pb.reshape(ntile, 1, _T), vtx_loc)


def _phase2(vf_loc, vf_full, vb_loc, vb_full, vl_loc, vl_full, flow_loc,
            flow_full):
    vloc, d = vf_loc.shape
    v = vf_full.shape[0]
    nblk = vloc // _VBLK
    ntile = v // _T
    lo, nt = _bounds(vb_loc.reshape(nblk, _VBLK), vb_full, ntile)
    grid_spec = pltpu.PrefetchScalarGridSpec(
        num_scalar_prefetch=2,
        grid=(nblk,),
        in_specs=[
            pl.BlockSpec((_VBLK, d), _blk),
            pl.BlockSpec((ntile, _T, d), _full3),
            pl.BlockSpec((_VBLK, 1), _blk),
            pl.BlockSpec((1, v), _full2),
            pl.BlockSpec((ntile, 1, _T), _full3),
            pl.BlockSpec((_VBLK, 1), _blk),
            pl.BlockSpec((1, v), _full2),
            pl.BlockSpec((ntile, 1, _T), _full3),
            pl.BlockSpec((_VBLK, 3), _blk),
            pl.BlockSpec((ntile, 4, _T), _full3),
        ],
        out_specs=pl.BlockSpec((_VBLK, 4), _blk),
        scratch_shapes=[
            pltpu.VMEM((ntile, _VBLK, _T), jnp.float32),
            pltpu.VMEM((ntile, _VBLK, _T), jnp.float32),
        ],
    )
    vlc = vl_full[:, 0]
    return pl.pallas_call(
        _knn2_body,
        grid_spec=grid_spec,
        out_shape=jax.ShapeDtypeStruct((vloc, 4), jnp.float32),
        compiler_params=_PARAMS,
    )(lo, nt, vf_loc, vf_full.reshape(ntile, _T, d), vb_loc[:, None],
      vb_full[None, :], vb_full.reshape(ntile, 1, _T), vl_loc,
      vlc[None, :], vlc.reshape(ntile, 1, _T), flow_loc,
      _ones_tiles(flow_full))


def _run_single(vtx, pts, vf, pf, vl, vb, pb):
    flow = _phase1(vf, pf, pts, vb, pb, vtx)
    return _phase2(vf, vf, vb, vb, vl, vl, flow, flow)


def _build(n_dev):
    if n_dev < 2:
        return jax.jit(_run_single)

    mesh = jax.sharding.Mesh(np.array(jax.devices()[:2]), ("x",))

    def _body(vtx, pts, vf, pf, vl, vb, pb):
        i = jax.lax.axis_index("x")
        half = vf.shape[0] // 2
        sl = lambda a: jax.lax.dynamic_slice_in_dim(a, i * half, half, axis=0)
        vf_h, vtx_h, vl_h, vb_h = sl(vf), sl(vtx), sl(vl), sl(vb)
        flow_h = _phase1(vf_h, pf, pts, vb_h, pb, vtx_h)
        flow_f = jax.lax.all_gather(flow_h, "x", axis=0, tiled=True)
        return _phase2(vf_h, vf, vb_h, vb, vl_h, vl, flow_h, flow_f)

    try:
        from jax.experimental.shard_map import shard_map
    except ImportError:
        shard_map = jax.shard_map

    sharded = shard_map(
        _body, mesh=mesh,
        in_specs=(P(), P(), P(), P(), P(), P(), P()),
        out_specs=P("x"),
        check_rep=False,
    )
    return jax.jit(sharded)


_CACHE = {}


def kernel(vtx, pts, vtx_feature, pts_feature, vis_logits, vtx_batch, pts_batch):
    n_dev = len(jax.devices())
    fn = _CACHE.get(n_dev)
    if fn is None:
        fn = _CACHE[n_dev] = _build(n_dev)
    return fn(vtx, pts, vtx_feature, pts_feature, vis_logits,
              vtx_batch, pts_batch)


# speedup vs baseline: 1.8995x; 1.1027x over previous
import jax
import jax.numpy as jnp
import numpy as np
from jax.experimental import pallas as pl
from jax.experimental.pallas import tpu as pltpu
from jax.sharding import PartitionSpec as P

_K = 8
_NB = 4
_NEG = -1e30
_VBLK = 256
_T = 1024
_BIG = 1 << 30


def _scan_topk(cbuf, lo, hi, rows):
    iota_t = jax.lax.broadcasted_iota(jnp.int32, (rows, _T), 1)
    for _ in range(_K):
        def maxstep(t, acc):
            return jnp.maximum(acc, jnp.max(cbuf[t], axis=1, keepdims=True))
        m = jax.lax.fori_loop(lo, hi, maxstep,
                              jnp.full((rows, 1), -jnp.inf, jnp.float32))

        def locstep(t, acc):
            io = iota_t + t * _T
            cand = jnp.min(jnp.where(cbuf[t] == m, io, _BIG),
                           axis=1, keepdims=True)
            return jnp.minimum(acc, cand)
        idx = jax.lax.fori_loop(lo, hi, locstep,
                                jnp.full((rows, 1), _BIG, jnp.int32))

        def constep(t, _):
            io = iota_t + t * _T
            cbuf[t] = jnp.where(io == idx, -jnp.inf, cbuf[t])
            return 0
        jax.lax.fori_loop(lo, hi, constep, 0)


def _accumulate(cbuf, sbuf, r4_ref, lo, hi, rows):
    zero = jnp.zeros((rows, 1), jnp.float32)

    def accstep(t, carry):
        d, n0, n1, n2 = carry
        w = jnp.where(cbuf[t] == -jnp.inf, sbuf[t], 0.0)
        r4 = r4_ref[t]
        d = d + jnp.sum(w, axis=1, keepdims=True)
        n0 = n0 + jnp.sum(w * r4[0:1, :], axis=1, keepdims=True)
        n1 = n1 + jnp.sum(w * r4[1:2, :], axis=1, keepdims=True)
        n2 = n2 + jnp.sum(w * r4[2:3, :], axis=1, keepdims=True)
        return (d, n0, n1, n2)

    return jax.lax.fori_loop(lo, hi, accstep, (zero, zero, zero, zero))


_DN = (((1,), (1,)), ((), ()))


def _knn1_body(lo_ref, nt_ref, vf_ref, pf_ref, pts1_ref, vb_ref, pb_ref,
               vtx_ref, out_ref, sbuf, cbuf):
    i = pl.program_id(0)
    lo = lo_ref[i]
    hi = lo + nt_ref[i]

    vf = vf_ref[...]
    rows = vf.shape[0]
    qsq = jnp.sum(vf * vf, axis=1, keepdims=True)
    qn = vf * (1.0 / (jnp.sqrt(qsq) + 1e-12))
    vb = vb_ref[...]

    def init_tile(t, _):
        pft = pf_ref[t]
        nsq = jnp.sum(pft * pft, axis=1, keepdims=True)
        pfn = pft * (1.0 / (jnp.sqrt(nsq) + 1e-12))
        S_t = jax.lax.dot_general(vf, pft, _DN,
                                  precision=jax.lax.Precision.HIGHEST,
                                  preferred_element_type=jnp.float32)
        C_t = jax.lax.dot_general(qn, pfn, _DN,
                                  preferred_element_type=jnp.float32)
        mask = vb == pb_ref[t]
        sbuf[t] = S_t
        cbuf[t] = jnp.where(mask, C_t, _NEG)
        return 0

    jax.lax.fori_loop(lo, hi, init_tile, 0)
    _scan_topk(cbuf, lo, hi, rows)
    den, num0, num1, num2 = _accumulate(cbuf, sbuf, pts1_ref, lo, hi, rows)

    vtx = vtx_ref[...]
    out_ref[:, 0:1] = num0 / den - vtx[:, 0:1]
    out_ref[:, 1:2] = num1 / den - vtx[:, 1:2]
    out_ref[:, 2:3] = num2 / den - vtx[:, 2:3]


def _pv_from(logits, batch, mx, mn):
    s = jax.nn.sigmoid(logits)
    out = jnp.zeros_like(s)
    for b in range(_NB):
        out = jnp.where(batch == b, (s - mn[b]) / (mx[b] - mn[b]), out)
    return out


def _knn2_body(lo_ref, nt_ref, vf_ref, vf3_ref, vb_ref, vbr_ref, vbr3_ref,
               vl_ref, vlr_ref, vlr3_ref, flow_ref, flow13_ref,
               out_ref, sbuf, cbuf):
    i = pl.program_id(0)
    lo = lo_ref[i]
    hi = lo + nt_ref[i]

    vf = vf_ref[...]
    rows = vf.shape[0]

    vbr = vbr_ref[...]
    sr = jax.nn.sigmoid(vlr_ref[...])
    mx, mn = [], []
    for b in range(_NB):
        inb = vbr == b
        mx.append(jnp.max(jnp.where(inb, sr, -jnp.inf), axis=1, keepdims=True))
        mn.append(jnp.min(jnp.where(inb, sr, jnp.inf), axis=1, keepdims=True))
    pv_blk = _pv_from(vl_ref[...], vb_ref[...], mx, mn)
    vis_blk = pv_blk >= 0.5

    qsq = jnp.sum(vf * vf, axis=1, keepdims=True)
    qn = vf * (1.0 / (jnp.sqrt(qsq) + 1e-12))
    vb = vb_ref[...]

    def init_tile(t, _):
        kt = vf3_ref[t]
        nsq = jnp.sum(kt * kt, axis=1, keepdims=True)
        kn = kt * (1.0 / (jnp.sqrt(nsq) + 1e-12))
        S_t = jax.lax.dot_general(vf, kt, _DN,
                                  precision=jax.lax.Precision.HIGHEST,
                                  preferred_element_type=jnp.float32)
        C_t = jax.lax.dot_general(qn, kn, _DN,
                                  preferred_element_type=jnp.float32)
        pv_t = _pv_from(vlr3_ref[t], vbr3_ref[t], mx, mn)
        mask = (vb == vbr3_ref[t]) & (pv_t >= 0.5)
        sbuf[t] = S_t
        cbuf[t] = jnp.where(mask, C_t, _NEG)
        return 0

    jax.lax.fori_loop(lo, hi, init_tile, 0)
    _scan_topk(cbuf, lo, hi, rows)
    den, num0, num1, num2 = _accumulate(cbuf, sbuf, flow13_ref, lo, hi, rows)

    flow = flow_ref[...]
    out_ref[:, 0:1] = jnp.where(vis_blk, flow[:, 0:1], num0 / den)
    out_ref[:, 1:2] = jnp.where(vis_blk, flow[:, 1:2], num1 / den)
    out_ref[:, 2:3] = jnp.where(vis_blk, flow[:, 2:3], num2 / den)
    out_ref[:, 3:4] = pv_blk


_PARAMS = pltpu.CompilerParams(dimension_semantics=("arbitrary",))


def _blk(i, *_):
    return (i, 0)


def _full2(i, *_):
    return (0, 0)


def _full3(i, *_):
    return (0, 0, 0)


def _bounds(q_batch_blocks, key_batch, n_tiles):
    b0 = q_batch_blocks[:, 0]
    b1 = q_batch_blocks[:, -1]
    start = jnp.searchsorted(key_batch, b0, side="left")
    end = jnp.searchsorted(key_batch, b1, side="right")
    lo = (start // _T).astype(jnp.int32)
    hi = ((end + _T - 1) // _T).astype(jnp.int32)
    ok = end > start
    lo = jnp.where(ok, lo, 0)
    nt = jnp.where(ok, jnp.minimum(hi - lo, n_tiles), 1).astype(jnp.int32)
    return lo, nt


def _ones_tiles(a):
    n = a.shape[0]
    r4 = jnp.concatenate([a.T, jnp.ones((1, n), jnp.float32)])
    return r4.reshape(4, n // _T, _T).transpose(1, 0, 2)


def _phase1(vf_loc, pf, pts, vb_loc, pb, vtx_loc):
    vloc, d = vf_loc.shape
    p = pf.shape[0]
    nblk = vloc // _VBLK
    ntile = p // _T
    lo, nt = _bounds(vb_loc.reshape(nblk, _VBLK), pb, ntile)
    grid_spec = pltpu.PrefetchScalarGridSpec(
        num_scalar_prefetch=2,
        grid=(nblk,),
        in_specs=[
            pl.BlockSpec((_VBLK, d), _blk),
            pl.BlockSpec((ntile, _T, d), _full3),
            pl.BlockSpec((ntile, 4, _T), _full3),
            pl.BlockSpec((_VBLK, 1), _blk),
            pl.BlockSpec((ntile, 1, _T), _full3),
            pl.BlockSpec((_VBLK, 3), _blk),
        ],
        out_specs=pl.BlockSpec((_VBLK, 3), _blk),
        scratch_shapes=[
            pltpu.VMEM((ntile, _VBLK, _T), jnp.float32),
            pltpu.VMEM((ntile, _VBLK, _T), jnp.float32),
        ],
    )
    return pl.pallas_call(
        _knn1_body,
        grid_spec=grid_spec,
        out_shape=jax.ShapeDtypeStruct((vloc, 3), jnp.float32),
        compiler_params=_PARAMS,
    )(lo, nt, vf_loc, pf.reshape(ntile, _T, d), _ones_tiles(pts),
      vb_loc[:, None], pb.reshape(ntile, 1, _T), vtx_loc)


def _phase2(vf_loc, vf_full, vb_loc, vb_full, vl_loc, vl_full, flow_loc,
            flow_full):
    vloc, d = vf_loc.shape
    v = vf_full.shape[0]
    nblk = vloc // _VBLK
    ntile = v // _T
    lo, nt = _bounds(vb_loc.reshape(nblk, _VBLK), vb_full, ntile)
    grid_spec = pltpu.PrefetchScalarGridSpec(
        num_scalar_prefetch=2,
        grid=(nblk,),
        in_specs=[
            pl.BlockSpec((_VBLK, d), _blk),
            pl.BlockSpec((ntile, _T, d), _full3),
            pl.BlockSpec((_VBLK, 1), _blk),
            pl.BlockSpec((1, v), _full2),
            pl.BlockSpec((ntile, 1, _T), _full3),
            pl.BlockSpec((_VBLK, 1), _blk),
            pl.BlockSpec((1, v), _full2),
            pl.BlockSpec((ntile, 1, _T), _full3),
            pl.BlockSpec((_VBLK, 3), _blk),
            pl.BlockSpec((ntile, 4, _T), _full3),
        ],
        out_specs=pl.BlockSpec((_VBLK, 4), _blk),
        scratch_shapes=[
            pltpu.VMEM((ntile, _VBLK, _T), jnp.float32),
            pltpu.VMEM((ntile, _VBLK, _T), jnp.float32),
        ],
    )
    vlc = vl_full[:, 0]
    return pl.pallas_call(
        _knn2_body,
        grid_spec=grid_spec,
        out_shape=jax.ShapeDtypeStruct((vloc, 4), jnp.float32),
        compiler_params=_PARAMS,
    )(lo, nt, vf_loc, vf_full.reshape(ntile, _T, d), vb_loc[:, None],
      vb_full[None, :], vb_full.reshape(ntile, 1, _T), vl_loc,
      vlc[None, :], vlc.reshape(ntile, 1, _T), flow_loc,
      _ones_tiles(flow_full))


def _run_single(vtx, pts, vf, pf, vl, vb, pb):
    flow = _phase1(vf, pf, pts, vb, pb, vtx)
    return _phase2(vf, vf, vb, vb, vl, vl, flow, flow)


def _build(n_dev):
    if n_dev < 2:
        return jax.jit(_run_single)

    mesh = jax.sharding.Mesh(np.array(jax.devices()[:2]), ("x",))

    def _body(vtx, pts, vf, pf, vl, vb, pb):
        i = jax.lax.axis_index("x")
        half = vf.shape[0] // 2
        sl = lambda a: jax.lax.dynamic_slice_in_dim(a, i * half, half, axis=0)
        vf_h, vtx_h, vl_h, vb_h = sl(vf), sl(vtx), sl(vl), sl(vb)
        flow_h = _phase1(vf_h, pf, pts, vb_h, pb, vtx_h)
        flow_f = jax.lax.all_gather(flow_h, "x", axis=0, tiled=True)
        return _phase2(vf_h, vf, vb_h, vb, vl_h, vl, flow_h, flow_f)

    try:
        from jax.experimental.shard_map import shard_map
    except ImportError:
        shard_map = jax.shard_map

    sharded = shard_map(
        _body, mesh=mesh,
        in_specs=(P(), P(), P(), P(), P(), P(), P()),
        out_specs=P("x"),
        check_rep=False,
    )
    return jax.jit(sharded)


_CACHE = {}


def kernel(vtx, pts, vtx_feature, pts_feature, vis_logits, vtx_batch, pts_batch):
    n_dev = len(jax.devices())
    fn = _CACHE.get(n_dev)
    if fn is None:
        fn = _CACHE[n_dev] = _build(n_dev)
    return fn(vtx, pts, vtx_feature, pts_feature, vis_logits,
              vtx_batch, pts_batch)
